# Initial kernel scaffold; baseline (speedup 1.0000x reference)
import jax
import jax.numpy as jnp
from jax.experimental import pallas as pl


def kernel(log_belief, semantic_map, action, w_red, b_red, w_dil, b_dil, w_exp, b_exp, w_k, b_k, w1, b1, w2, b2):
    raise NotImplementedError("write your pallas kernel here")


# speedup vs baseline: 91.5384x; 91.5384x over previous
import functools

import jax
import jax.numpy as jnp
from jax.experimental import pallas as pl

_K = 5
_R = 32
_NEG = float("-inf")


def _shift_cols(a, dx, fill_value):
    if dx == 0:
        return a
    fill = jnp.full(a.shape[:-1] + (abs(dx),), fill_value, a.dtype)
    if dx > 0:
        return jnp.concatenate([a[..., dx:], fill], axis=-1)
    return jnp.concatenate([fill, a[..., :a.shape[-1] + dx]], axis=-1)


def _mm(wm, x2):
    return jax.lax.dot_general(
        wm.astype(jnp.bfloat16), x2.astype(jnp.bfloat16),
        (((1,), (0,)), ((), ())), preferred_element_type=jnp.float32)


def _band_kernel(lb_ref, sm_ref, act_ref, wred_ref, bred_ref, wdil_ref,
                 bdil_ref, wexp_ref, bexp_ref, wk_ref, bk_ref, w1_ref, b1_ref,
                 w2_ref, b2_ref, out_ref, *, h, w, kk):
    b = pl.program_id(1)
    row0 = b * _R

    c = sm_ref.shape[1]
    xb = sm_ref[0, :, pl.ds(row0, _R + 8), :]

    f1 = jnp.maximum(_mm(wred_ref[...], xb.reshape(c, (_R + 8) * w))
                     + bred_ref[...], 0.0)
    f1 = f1.reshape(c, _R + 8, w)
    rid = jax.lax.broadcasted_iota(jnp.int32, (1, _R + 8, w), 1) + (row0 - 4)
    f1 = jnp.where((rid >= 0) & (rid < h), f1, 0.0)

    m = (_R + 4) * w
    acc = jnp.zeros((c, m), jnp.float32)
    for t, (dy, dx) in enumerate((dy, dx) for dy in (-2, 0, 2)
                                 for dx in (-2, 0, 2)):
        sl = _shift_cols(f1[:, 2 + dy:2 + dy + _R + 4, :], dx, 0.0)
        acc = acc + _mm(wdil_ref[t], sl.reshape(c, m))
    f2 = jnp.maximum(acc + bdil_ref[...], 0.0)

    f3 = _mm(wexp_ref[...], f2) + bexp_ref[...]
    sm_c = xb[:, 2:2 + _R + 4, :].reshape(c, m)
    feat = jnp.maximum(sm_c + f3, 0.0)

    a_col = act_ref[0, 0, :].reshape(-1, 1)
    hcol = jnp.maximum(_mm(w1_ref[...], a_col) + b1_ref[...], 0.0)
    lvec = _mm(w2_ref[...], hcol) + b2_ref[...]

    logits = _mm(wk_ref[...], feat) + bk_ref[...] + lvec
    m25 = jnp.max(logits, axis=0, keepdims=True)
    lse = m25 + jnp.log(jnp.sum(jnp.exp(logits - m25), axis=0, keepdims=True))
    z = logits - lse

    lb = lb_ref[0, 0, pl.ds(row0, _R + 4), :].reshape(1, m)
    contrib = (z + lb).reshape(kk, _R + 4, w)
    rid2 = jax.lax.broadcasted_iota(jnp.int32, (1, _R + 4, w), 1) + (row0 - 2)
    contrib = jnp.where((rid2 >= 0) & (rid2 < h), contrib, _NEG)

    terms = []
    for i in range(_K):
        for j in range(_K):
            di, dj = i - 2, j - 2
            t2 = contrib[i * _K + j, 2 - di:2 - di + _R, :]
            terms.append(_shift_cols(t2, -dj, _NEG))
    mx = functools.reduce(jnp.maximum, terms)
    s = functools.reduce(lambda u, v: u + v,
                         (jnp.exp(t - mx) for t in terms))
    out_ref[0, 0] = mx + jnp.log(s)


def kernel(log_belief, semantic_map, action, w_red, b_red, w_dil, b_dil,
           w_exp, b_exp, w_k, b_k, w1, b1, w2, b2):
    n, cin, h, w = log_belief.shape
    mapc = semantic_map.shape[1]
    hid = w_red.shape[0]
    kk = w_k.shape[0]
    aemb = action.shape[1]

    sm_pad = jnp.pad(semantic_map, ((0, 0), (0, 0), (4, 4), (0, 0)))
    lb_pad = jnp.pad(log_belief, ((0, 0), (0, 0), (2, 2), (0, 0)))
    wred_m = w_red.reshape(hid, mapc)
    wdil_m = jnp.transpose(w_dil, (2, 3, 0, 1)).reshape(9, hid, hid)
    wexp_m = w_exp.reshape(mapc, hid)
    wk_m = w_k.reshape(kk, mapc)
    w1t = w1.T
    w2t = w2.T
    col = lambda v: v.reshape(-1, 1)

    nb = h // _R
    full = lambda i, b: (i, 0, 0, 0)
    zero2 = lambda i, b: (0, 0)
    zero3 = lambda i, b: (0, 0, 0)

    return pl.pallas_call(
        functools.partial(_band_kernel, h=h, w=w, kk=kk),
        grid=(n, nb),
        in_specs=[
            pl.BlockSpec((1, 1, h + 4, w), full),
            pl.BlockSpec((1, mapc, h + 8, w), full),
            pl.BlockSpec((1, 1, aemb), lambda i, b: (i, 0, 0)),
            pl.BlockSpec((hid, mapc), zero2),
            pl.BlockSpec((hid, 1), zero2),
            pl.BlockSpec((9, hid, hid), zero3),
            pl.BlockSpec((hid, 1), zero2),
            pl.BlockSpec((mapc, hid), zero2),
            pl.BlockSpec((mapc, 1), zero2),
            pl.BlockSpec((kk, mapc), zero2),
            pl.BlockSpec((kk, 1), zero2),
            pl.BlockSpec((hid, aemb), zero2),
            pl.BlockSpec((hid, 1), zero2),
            pl.BlockSpec((kk, hid), zero2),
            pl.BlockSpec((kk, 1), zero2),
        ],
        out_specs=pl.BlockSpec((1, 1, _R, w), lambda i, b: (i, 0, b, 0)),
        out_shape=jax.ShapeDtypeStruct((n, cin, h, w), jnp.float32),
    )(lb_pad, sm_pad, action.reshape(n, 1, aemb), wred_m, col(b_red), wdil_m, col(b_dil),
      wexp_m, col(b_exp), wk_m, col(b_k), w1t, col(b1), w2t, col(b2))


# speedup vs baseline: 148.3961x; 1.6211x over previous
import functools

import jax
import jax.numpy as jnp
from jax.experimental import pallas as pl

_K = 5
_R = 32
_NEG = float("-inf")


def _shift_cols(a, dx, fill_value):
    if dx == 0:
        return a
    fill = jnp.full(a.shape[:-1] + (abs(dx),), fill_value, a.dtype)
    if dx > 0:
        return jnp.concatenate([a[..., dx:], fill], axis=-1)
    return jnp.concatenate([fill, a[..., :a.shape[-1] + dx]], axis=-1)


def _mm(wb, xb):
    return jax.lax.dot_general(wb, xb, (((1,), (0,)), ((), ())),
                               preferred_element_type=jnp.float32)


def _band_kernel(lbp_ref, lbm_ref, lbn_ref, smp_ref, smm_ref, smn_ref,
                 act_ref, wred_ref, bred_ref, wdil_ref, bdil_ref, wexp_ref,
                 bexp_ref, wk_ref, bk_ref, w1_ref, b1_ref, w2_ref, b2_ref,
                 out_ref, *, h, w, kk):
    b = pl.program_id(1)
    row0 = b * _R
    m8 = (_R + 8) * w
    m4 = (_R + 4) * w

    x8 = jnp.concatenate(
        [smp_ref[0, :, (_R - 4) * w:], smm_ref[0], smn_ref[0, :, :4 * w]],
        axis=1)

    f1 = jnp.maximum(_mm(wred_ref[...], x8.astype(jnp.bfloat16))
                     + bred_ref[...], 0.0)
    rid8 = jax.lax.broadcasted_iota(jnp.int32, (1, m8), 1) // w + (row0 - 4)
    f1 = jnp.where((rid8 >= 0) & (rid8 < h), f1, 0.0).astype(jnp.bfloat16)

    cid8 = jax.lax.broadcasted_iota(jnp.int32, (1, m8), 1) % w
    zer = jnp.zeros_like(f1[:, :2])
    f1l = jnp.where(cid8 >= 2, jnp.concatenate([zer, f1[:, :-2]], axis=1),
                    jnp.bfloat16(0))
    f1r = jnp.where(cid8 < w - 2, jnp.concatenate([f1[:, 2:], zer], axis=1),
                    jnp.bfloat16(0))

    acc = None
    for ky in range(3):
        base = (2 + (ky - 1) * 2) * w
        for kx, src in ((0, f1l), (1, f1), (2, f1r)):
            p = _mm(wdil_ref[ky * 3 + kx], src[:, base:base + m4])
            acc = p if acc is None else acc + p
    f2 = jnp.maximum(acc + bdil_ref[...], 0.0).astype(jnp.bfloat16)

    f3 = _mm(wexp_ref[...], f2) + bexp_ref[...]
    feat = jnp.maximum(x8[:, 2 * w:2 * w + m4] + f3, 0.0)

    a_col = act_ref[0, 0, :].reshape(-1, 1).astype(jnp.bfloat16)
    hcol = jnp.maximum(_mm(w1_ref[...], a_col) + b1_ref[...], 0.0)
    lvec = _mm(w2_ref[...], hcol.astype(jnp.bfloat16)) + b2_ref[...]

    logits = _mm(wk_ref[...], feat.astype(jnp.bfloat16)) \
        + bk_ref[...] + lvec
    m25 = jnp.max(logits, axis=0, keepdims=True)
    lse = m25 + jnp.log(jnp.sum(jnp.exp(logits - m25), axis=0, keepdims=True))
    z = logits - lse

    lb = jnp.concatenate(
        [lbp_ref[0, :, (_R - 2) * w:], lbm_ref[0], lbn_ref[0, :, :2 * w]],
        axis=1)
    contrib = z + lb
    rid4 = jax.lax.broadcasted_iota(jnp.int32, (1, m4), 1) // w + (row0 - 2)
    contrib = jnp.where((rid4 >= 0) & (rid4 < h), contrib, _NEG)
    c3 = contrib.reshape(kk, _R + 4, w)

    terms = []
    for i in range(_K):
        for j in range(_K):
            di, dj = i - 2, j - 2
            t2 = c3[i * _K + j, 2 - di:2 - di + _R, :]
            terms.append(_shift_cols(t2, -dj, _NEG))
    mx = functools.reduce(jnp.maximum, terms)
    s = functools.reduce(lambda u, v: u + v,
                         (jnp.exp(t - mx) for t in terms))
    out_ref[0, 0] = mx + jnp.log(s)


def kernel(log_belief, semantic_map, action, w_red, b_red, w_dil, b_dil,
           w_exp, b_exp, w_k, b_k, w1, b1, w2, b2):
    n, cin, h, w = log_belief.shape
    mapc = semantic_map.shape[1]
    hid = w_red.shape[0]
    kk = w_k.shape[0]
    aemb = action.shape[1]
    nb = h // _R

    sm2 = semantic_map.reshape(n, mapc, h * w)
    lb2 = log_belief.reshape(n, cin, h * w)
    bf = lambda v: v.astype(jnp.bfloat16)
    wred_m = bf(w_red.reshape(hid, mapc))
    wdil_m = bf(jnp.transpose(w_dil, (2, 3, 0, 1)).reshape(9, hid, hid))
    wexp_m = bf(w_exp.reshape(mapc, hid))
    wk_m = bf(w_k.reshape(kk, mapc))
    w1t = bf(w1.T)
    w2t = bf(w2.T)
    col = lambda v: v.reshape(-1, 1)

    prv = lambda i, b: (i, 0, jnp.maximum(b - 1, 0))
    mid = lambda i, b: (i, 0, b)
    nxt = lambda i, b: (i, 0, jnp.minimum(b + 1, nb - 1))
    zero2 = lambda i, b: (0, 0)
    zero3 = lambda i, b: (0, 0, 0)
    lb_spec = lambda im: pl.BlockSpec((1, cin, _R * w), im)
    sm_spec = lambda im: pl.BlockSpec((1, mapc, _R * w), im)

    return pl.pallas_call(
        functools.partial(_band_kernel, h=h, w=w, kk=kk),
        grid=(n, nb),
        in_specs=[
            lb_spec(prv), lb_spec(mid), lb_spec(nxt),
            sm_spec(prv), sm_spec(mid), sm_spec(nxt),
            pl.BlockSpec((1, 1, aemb), lambda i, b: (i, 0, 0)),
            pl.BlockSpec((hid, mapc), zero2),
            pl.BlockSpec((hid, 1), zero2),
            pl.BlockSpec((9, hid, hid), zero3),
            pl.BlockSpec((hid, 1), zero2),
            pl.BlockSpec((mapc, hid), zero2),
            pl.BlockSpec((mapc, 1), zero2),
            pl.BlockSpec((kk, mapc), zero2),
            pl.BlockSpec((kk, 1), zero2),
            pl.BlockSpec((hid, aemb), zero2),
            pl.BlockSpec((hid, 1), zero2),
            pl.BlockSpec((kk, hid), zero2),
            pl.BlockSpec((kk, 1), zero2),
        ],
        out_specs=pl.BlockSpec((1, 1, _R, w), lambda i, b: (i, 0, b, 0)),
        out_shape=jax.ShapeDtypeStruct((n, cin, h, w), jnp.float32),
    )(lb2, lb2, lb2, sm2, sm2, sm2, action.reshape(n, 1, aemb), wred_m,
      col(b_red), wdil_m, col(b_dil), wexp_m, col(b_exp), wk_m, col(b_k),
      w1t, col(b1), w2t, col(b2))


# speedup vs baseline: 149.6544x; 1.0085x over previous
import functools

import jax
import jax.numpy as jnp
from jax.experimental import pallas as pl
from jax.experimental.pallas import tpu as pltpu

_K = 5
_R = 64
_NEG = float("-inf")


def _shift_cols(a, dx, fill_value):
    if dx == 0:
        return a
    fill = jnp.full(a.shape[:-1] + (abs(dx),), fill_value, a.dtype)
    if dx > 0:
        return jnp.concatenate([a[..., dx:], fill], axis=-1)
    return jnp.concatenate([fill, a[..., :a.shape[-1] + dx]], axis=-1)


def _mm(wb, xb):
    return jax.lax.dot_general(wb, xb, (((1,), (0,)), ((), ())),
                               preferred_element_type=jnp.float32)


def _band_kernel(lbp_ref, lbm_ref, lbn_ref, smp_ref, smm_ref, smn_ref,
                 act_ref, wred_ref, bred_ref, wdil_ref, bdil_ref, wexp_ref,
                 bexp_ref, wk_ref, bk_ref, w1_ref, b1_ref, w2_ref, b2_ref,
                 out_ref, lvec_ref, *, h, w, kk):
    b = pl.program_id(1)
    row0 = b * _R
    m8 = (_R + 8) * w
    m4 = (_R + 4) * w

    @pl.when(b == 0)
    def _():
        a_col = act_ref[0, 0, :].reshape(-1, 1).astype(jnp.bfloat16)
        hcol = jnp.maximum(_mm(w1_ref[...], a_col) + b1_ref[...], 0.0)
        lvec_ref[...] = _mm(w2_ref[...], hcol.astype(jnp.bfloat16)) \
            + b2_ref[...]

    x8 = jnp.concatenate(
        [smp_ref[0, :, (_R - 4) * w:].astype(jnp.bfloat16),
         smm_ref[0].astype(jnp.bfloat16),
         smn_ref[0, :, :4 * w].astype(jnp.bfloat16)],
        axis=1)

    f1 = jnp.maximum(_mm(wred_ref[...], x8) + bred_ref[...], 0.0)
    rid8 = jax.lax.broadcasted_iota(jnp.int32, (1, m8), 1) // w + (row0 - 4)
    f1 = jnp.where((rid8 >= 0) & (rid8 < h), f1, 0.0).astype(jnp.bfloat16)

    cid8 = jax.lax.broadcasted_iota(jnp.int32, (1, m8), 1) % w
    zer = jnp.zeros_like(f1[:, :2])
    f1l = jnp.where(cid8 >= 2, jnp.concatenate([zer, f1[:, :-2]], axis=1),
                    jnp.bfloat16(0))
    f1r = jnp.where(cid8 < w - 2, jnp.concatenate([f1[:, 2:], zer], axis=1),
                    jnp.bfloat16(0))

    acc = None
    for ky in range(3):
        base = (2 + (ky - 1) * 2) * w
        for kx, src in ((0, f1l), (1, f1), (2, f1r)):
            p = _mm(wdil_ref[ky * 3 + kx], src[:, base:base + m4])
            acc = p if acc is None else acc + p
    f2 = jnp.maximum(acc + bdil_ref[...], 0.0).astype(jnp.bfloat16)

    f3 = _mm(wexp_ref[...], f2) + bexp_ref[...]
    feat = jnp.maximum(x8[:, 2 * w:2 * w + m4] + f3, 0.0)

    logits = _mm(wk_ref[...], feat.astype(jnp.bfloat16)) \
        + bk_ref[...] + lvec_ref[...]
    m25 = jnp.max(logits, axis=0, keepdims=True)
    lse = m25 + jnp.log(jnp.sum(jnp.exp(logits - m25), axis=0, keepdims=True))

    lb = jnp.concatenate(
        [lbp_ref[0, :, (_R - 2) * w:], lbm_ref[0], lbn_ref[0, :, :2 * w]],
        axis=1)
    contrib = logits - (lse - lb)
    rid4 = jax.lax.broadcasted_iota(jnp.int32, (1, m4), 1) // w + (row0 - 2)
    contrib = jnp.where((rid4 >= 0) & (rid4 < h), contrib, _NEG)
    c3 = contrib.reshape(kk, _R + 4, w)

    terms = []
    for i in range(_K):
        for j in range(_K):
            di, dj = i - 2, j - 2
            t2 = c3[i * _K + j, 2 - di:2 - di + _R, :]
            terms.append(_shift_cols(t2, -dj, _NEG))
    mx = functools.reduce(jnp.maximum, terms)
    s = functools.reduce(lambda u, v: u + v,
                         (jnp.exp(t - mx) for t in terms))
    out_ref[0, 0] = mx + jnp.log(s)


def kernel(log_belief, semantic_map, action, w_red, b_red, w_dil, b_dil,
           w_exp, b_exp, w_k, b_k, w1, b1, w2, b2):
    n, cin, h, w = log_belief.shape
    mapc = semantic_map.shape[1]
    hid = w_red.shape[0]
    kk = w_k.shape[0]
    aemb = action.shape[1]
    nb = h // _R

    sm2 = semantic_map.reshape(n, mapc, h * w)
    lb2 = log_belief.reshape(n, cin, h * w)
    bf = lambda v: v.astype(jnp.bfloat16)
    wred_m = bf(w_red.reshape(hid, mapc))
    wdil_m = bf(jnp.transpose(w_dil, (2, 3, 0, 1)).reshape(9, hid, hid))
    wexp_m = bf(w_exp.reshape(mapc, hid))
    wk_m = bf(w_k.reshape(kk, mapc))
    w1t = bf(w1.T)
    w2t = bf(w2.T)
    col = lambda v: v.reshape(-1, 1)

    prv = lambda i, b: (i, 0, jnp.maximum(b - 1, 0))
    mid = lambda i, b: (i, 0, b)
    nxt = lambda i, b: (i, 0, jnp.minimum(b + 1, nb - 1))
    zero2 = lambda i, b: (0, 0)
    zero3 = lambda i, b: (0, 0, 0)
    lb_spec = lambda im: pl.BlockSpec((1, cin, _R * w), im)
    sm_spec = lambda im: pl.BlockSpec((1, mapc, _R * w), im)

    return pl.pallas_call(
        functools.partial(_band_kernel, h=h, w=w, kk=kk),
        grid=(n, nb),
        in_specs=[
            lb_spec(prv), lb_spec(mid), lb_spec(nxt),
            sm_spec(prv), sm_spec(mid), sm_spec(nxt),
            pl.BlockSpec((1, 1, aemb), lambda i, b: (i, 0, 0)),
            pl.BlockSpec((hid, mapc), zero2),
            pl.BlockSpec((hid, 1), zero2),
            pl.BlockSpec((9, hid, hid), zero3),
            pl.BlockSpec((hid, 1), zero2),
            pl.BlockSpec((mapc, hid), zero2),
            pl.BlockSpec((mapc, 1), zero2),
            pl.BlockSpec((kk, mapc), zero2),
            pl.BlockSpec((kk, 1), zero2),
            pl.BlockSpec((hid, aemb), zero2),
            pl.BlockSpec((hid, 1), zero2),
            pl.BlockSpec((kk, hid), zero2),
            pl.BlockSpec((kk, 1), zero2),
        ],
        out_specs=pl.BlockSpec((1, 1, _R, w), lambda i, b: (i, 0, b, 0)),
        out_shape=jax.ShapeDtypeStruct((n, cin, h, w), jnp.float32),
        scratch_shapes=[pltpu.VMEM((kk, 1), jnp.float32)],
    )(lb2, lb2, lb2, sm2, sm2, sm2, action.reshape(n, 1, aemb), wred_m,
      col(b_red), wdil_m, col(b_dil), wexp_m, col(b_exp), wk_m, col(b_k),
      w1t, col(b1), w2t, col(b2))
